# baseline (speedup 1.0000x reference)
import functools

import jax
import jax.numpy as jnp
from jax import lax
from jax.experimental import pallas as pl
from jax.experimental.pallas import tpu as pltpu
from jax.experimental.pallas import tpu_sc as plsc

VOCAB = 100000
DIM = 128
B = 16384
K = 20
J = K + 1
OUTW = 32
LANES = 16
NSEG = DIM // LANES

NC = 2
NS = 16
NW = NC * NS

GCH = 112


def _sc_body(cb, rpw, c_hbm, np_hbm, cen_hbm, ctx_hbm, out_hbm,
             ce0, cx0, sem0,
             ce1, cx1, sem1,
             cidx_all, npidx_all,
             tr0_v, tr1_v, out_v):
    nchunk = rpw // cb
    ng = (cb * J) // GCH
    wid = lax.axis_index("s") * NC + lax.axis_index("c")
    iota = lax.iota(jnp.int32, LANES)
    zero = jnp.zeros((LANES,), jnp.float32)
    pltpu.sync_copy(c_hbm.at[pl.ds(wid * rpw, rpw)], cidx_all)
    pltpu.sync_copy(np_hbm.at[pl.ds(wid * rpw * J, rpw * J)], npidx_all)
    for r in range(J, 2 * LANES):
        tr0_v[pl.ds(r * LANES, LANES)] = zero
        tr1_v[pl.ds(r * LANES, LANES)] = zero

    bufs = ((ce0, cx0, sem0),
            (ce1, cx1, sem1))
    rowbase = iota * LANES

    def issue(t, buf):
        ce_v, cx_v, sem = buf
        pltpu.async_copy(
            cen_hbm.at[cidx_all.at[pl.ds(t * cb, cb)]], ce_v, sem
        )
        for g in range(ng):
            pltpu.async_copy(
                ctx_hbm.at[npidx_all.at[pl.ds(t * cb * J + g * GCH, GCH)]],
                cx_v.at[pl.ds(g * GCH, GCH)],
                sem,
            )

    def wait(t, buf):
        ce_v, cx_v, sem = buf
        pltpu.make_async_copy(
            cen_hbm.at[cidx_all.at[pl.ds(t * cb, cb)]], ce_v, sem
        ).wait()
        pltpu.make_async_copy(
            ctx_hbm.at[npidx_all.at[pl.ds(t * cb * J, cb * J)]], cx_v, sem
        ).wait()

    def compute(t, buf):
        ce_v, cx_v, _ = buf

        def one_row(b, tr_v):
            ce = [ce_v[b, pl.ds(LANES * i, LANES)] for i in range(NSEG)]

            def ctx_row(j):
                r = b * J + j
                return [
                    cx_v[r, pl.ds(LANES * i, LANES)] for i in range(NSEG)
                ]

            cur = ctx_row(0)
            for j in range(J):
                nxt = ctx_row(j + 1) if j + 1 < J else None
                prods = [ce[i] * cur[i] for i in range(NSEG)]
                while len(prods) > 1:
                    prods = [
                        prods[q] + prods[q + 1]
                        for q in range(0, len(prods), 2)
                    ]
                tr_v[pl.ds(j * LANES, LANES)] = prods[0]
                cur = nxt
            for h in range(2):
                cols = [
                    plsc.load_gather(
                        tr_v, [rowbase + (h * LANES * LANES + m)]
                    )
                    for m in range(LANES)
                ]
                while len(cols) > 1:
                    cols = [
                        cols[q] + cols[q + 1]
                        for q in range(0, len(cols), 2)
                    ]
                out_v[b, pl.ds(LANES * h, LANES)] = cols[0]

        def row_body(g2, c2):
            one_row(2 * g2, tr0_v)
            one_row(2 * g2 + 1, tr1_v)
            return c2

        lax.fori_loop(0, cb // 2, row_body, 0, unroll=2)
        base = wid * rpw + t * cb
        pltpu.sync_copy(out_v, out_hbm.at[pl.ds(base, cb)])

    issue(0, bufs[0])

    def pair_body(g, carry):
        t0 = 2 * g
        issue(t0 + 1, bufs[1])
        wait(t0, bufs[0])
        compute(t0, bufs[0])
        issue(lax.rem(t0 + 2, nchunk), bufs[0])
        wait(t0 + 1, bufs[1])
        compute(t0 + 1, bufs[1])
        return carry

    lax.fori_loop(0, nchunk // 2, pair_body, 0, unroll=False)
    wait(0, bufs[0])


def _make_sc_dots(b_total, cb, interpret=False):
    rpw = b_total // NW
    buf = [
        pltpu.VMEM((cb, DIM), jnp.float32),
        pltpu.VMEM((cb * J, DIM), jnp.float32),
        pltpu.SemaphoreType.DMA,
    ]
    return functools.partial(
        pl.kernel,
        out_type=jax.ShapeDtypeStruct((b_total, OUTW), jnp.float32),
        mesh=plsc.VectorSubcoreMesh(
            core_axis_name="c", subcore_axis_name="s",
            num_cores=NC, num_subcores=NS,
        ),
        scratch_types=buf + buf + [
            pltpu.VMEM((rpw,), jnp.int32),
            pltpu.VMEM((rpw * J,), jnp.int32),
            pltpu.VMEM((2 * LANES * LANES,), jnp.float32),
            pltpu.VMEM((2 * LANES * LANES,), jnp.float32),
            pltpu.VMEM((cb, OUTW), jnp.float32),
        ],
        compiler_params=pltpu.CompilerParams(needs_layout_passes=False),
        interpret=interpret,
    )(functools.partial(_sc_body, cb, rpw))


def _loss_body(bk, x_ref, o_ref):
    x = x_ref[...]
    col = lax.broadcasted_iota(jnp.int32, x.shape, 1) % OUTW
    sign = jnp.where(col == K, 1.0, -1.0).astype(jnp.float32)
    w = jnp.where(
        col == K, 1.0 / bk, jnp.where(col < K, 1.0 / (bk * K), 0.0)
    ).astype(jnp.float32)
    t = sign * x
    ls = jnp.minimum(t, 0.0) - jnp.log1p(jnp.exp(-jnp.abs(t)))
    o_ref[0, 0] = -jnp.sum(w * ls)


def _loss_from_dots(dots, interpret=False):
    bk = dots.shape[0]
    x = dots.reshape(bk * OUTW // 128, 128)
    out = pl.pallas_call(
        functools.partial(_loss_body, bk),
        out_shape=jax.ShapeDtypeStruct((1, 1), jnp.float32),
        out_specs=pl.BlockSpec(memory_space=pltpu.SMEM),
        interpret=interpret,
    )(x)
    return out[0, 0]


@jax.jit
def kernel(c, p, n, center_w, context_w):
    c = c.astype(jnp.int32)
    np_idx = jnp.concatenate(
        [n.astype(jnp.int32), p.astype(jnp.int32)[:, None]], axis=1
    ).reshape(-1)
    dots = _make_sc_dots(B, 16)(c, np_idx, center_w, context_w)
    return _loss_from_dots(dots)


# speedup vs baseline: 1.0721x; 1.0721x over previous
import functools

import jax
import jax.numpy as jnp
from jax import lax
from jax.experimental import pallas as pl
from jax.experimental.pallas import tpu as pltpu
from jax.experimental.pallas import tpu_sc as plsc

VOCAB = 100000
DIM = 128
B = 16384
K = 20
J = K + 1
OUTW = 32
LANES = 16
NSEG = DIM // LANES

NC = 2
NS = 16
NW = NC * NS

GCH = 112


def _sc_body(cb, rpw, c_hbm, np_hbm, cen_hbm, ctx_hbm, out_hbm,
             ce0, cx0, sem0,
             ce1, cx1, sem1,
             cidx_all, npidx_all,
             tr0_v, tr1_v, out_v):
    nchunk = rpw // cb
    ng = (cb * J) // GCH
    wid = lax.axis_index("s") * NC + lax.axis_index("c")
    iota = lax.iota(jnp.int32, LANES)
    zero = jnp.zeros((LANES,), jnp.float32)
    pltpu.sync_copy(c_hbm.at[pl.ds(wid * rpw, rpw)], cidx_all)
    pltpu.sync_copy(np_hbm.at[pl.ds(wid * rpw * J, rpw * J)], npidx_all)
    for r in range(J, 2 * LANES):
        tr0_v[pl.ds(r * LANES, LANES)] = zero
        tr1_v[pl.ds(r * LANES, LANES)] = zero

    bufs = ((ce0, cx0, sem0),
            (ce1, cx1, sem1))
    rowbase = iota * LANES

    def issue(t, buf):
        ce_v, cx_v, sem = buf
        pltpu.async_copy(
            cen_hbm.at[cidx_all.at[pl.ds(t * cb, cb)]], ce_v, sem
        )
        for g in range(ng):
            pltpu.async_copy(
                ctx_hbm.at[npidx_all.at[pl.ds(t * cb * J + g * GCH, GCH)]],
                cx_v.at[pl.ds(g * GCH, GCH)],
                sem,
            )

    def wait(t, buf):
        ce_v, cx_v, sem = buf
        pltpu.make_async_copy(
            cen_hbm.at[cidx_all.at[pl.ds(t * cb, cb)]], ce_v, sem
        ).wait()
        pltpu.make_async_copy(
            ctx_hbm.at[npidx_all.at[pl.ds(t * cb * J, cb * J)]], cx_v, sem
        ).wait()

    def compute(t, buf):
        ce_v, cx_v, _ = buf

        def one_row(b, tr_v):
            ce = [ce_v[b, pl.ds(LANES * i, LANES)] for i in range(NSEG)]

            def ctx_row(j):
                r = b * J + j
                return [
                    cx_v[r, pl.ds(LANES * i, LANES)] for i in range(NSEG)
                ]

            cur = ctx_row(0)
            for j in range(J):
                nxt = ctx_row(j + 1) if j + 1 < J else None
                prods = [ce[i] * cur[i] for i in range(NSEG)]
                while len(prods) > 1:
                    prods = [
                        prods[q] + prods[q + 1]
                        for q in range(0, len(prods), 2)
                    ]
                tr_v[pl.ds(j * LANES, LANES)] = prods[0]
                cur = nxt
            for h in range(2):
                cols = [
                    plsc.load_gather(
                        tr_v, [rowbase + (h * LANES * LANES + m)]
                    )
                    for m in range(LANES)
                ]
                while len(cols) > 1:
                    cols = [
                        cols[q] + cols[q + 1]
                        for q in range(0, len(cols), 2)
                    ]
                out_v[b, pl.ds(LANES * h, LANES)] = cols[0]

        def row_body(g2, c2):
            one_row(2 * g2, tr0_v)
            one_row(2 * g2 + 1, tr1_v)
            return c2

        lax.fori_loop(0, cb // 2, row_body, 0, unroll=False)
        base = wid * rpw + t * cb
        pltpu.sync_copy(out_v, out_hbm.at[pl.ds(base, cb)])

    issue(0, bufs[0])

    def pair_body(g, carry):
        t0 = 2 * g
        issue(t0 + 1, bufs[1])
        wait(t0, bufs[0])
        compute(t0, bufs[0])
        issue(lax.rem(t0 + 2, nchunk), bufs[0])
        wait(t0 + 1, bufs[1])
        compute(t0 + 1, bufs[1])
        return carry

    lax.fori_loop(0, nchunk // 2, pair_body, 0, unroll=False)
    wait(0, bufs[0])


def _make_sc_dots(b_total, cb, interpret=False):
    rpw = b_total // NW
    buf = [
        pltpu.VMEM((cb, DIM), jnp.float32),
        pltpu.VMEM((cb * J, DIM), jnp.float32),
        pltpu.SemaphoreType.DMA,
    ]
    return functools.partial(
        pl.kernel,
        out_type=jax.ShapeDtypeStruct((b_total, OUTW), jnp.float32),
        mesh=plsc.VectorSubcoreMesh(
            core_axis_name="c", subcore_axis_name="s",
            num_cores=NC, num_subcores=NS,
        ),
        scratch_types=buf + buf + [
            pltpu.VMEM((rpw,), jnp.int32),
            pltpu.VMEM((rpw * J,), jnp.int32),
            pltpu.VMEM((2 * LANES * LANES,), jnp.float32),
            pltpu.VMEM((2 * LANES * LANES,), jnp.float32),
            pltpu.VMEM((cb, OUTW), jnp.float32),
        ],
        compiler_params=pltpu.CompilerParams(needs_layout_passes=False),
        interpret=interpret,
    )(functools.partial(_sc_body, cb, rpw))


def _loss_body(bk, x_ref, o_ref):
    x = x_ref[...]
    col = lax.broadcasted_iota(jnp.int32, x.shape, 1) % OUTW
    sign = jnp.where(col == K, 1.0, -1.0).astype(jnp.float32)
    w = jnp.where(
        col == K, 1.0 / bk, jnp.where(col < K, 1.0 / (bk * K), 0.0)
    ).astype(jnp.float32)
    t = sign * x
    ls = jnp.minimum(t, 0.0) - jnp.log1p(jnp.exp(-jnp.abs(t)))
    o_ref[0, 0] = -jnp.sum(w * ls)


def _loss_from_dots(dots, interpret=False):
    bk = dots.shape[0]
    x = dots.reshape(bk * OUTW // 128, 128)
    out = pl.pallas_call(
        functools.partial(_loss_body, bk),
        out_shape=jax.ShapeDtypeStruct((1, 1), jnp.float32),
        out_specs=pl.BlockSpec(memory_space=pltpu.SMEM),
        interpret=interpret,
    )(x)
    return out[0, 0]


@jax.jit
def kernel(c, p, n, center_w, context_w):
    c = c.astype(jnp.int32)
    np_idx = jnp.concatenate(
        [n.astype(jnp.int32), p.astype(jnp.int32)[:, None]], axis=1
    ).reshape(-1)
    dots = _make_sc_dots(B, 16)(c, np_idx, center_w, context_w)
    return _loss_from_dots(dots)


# speedup vs baseline: 1.1117x; 1.0369x over previous
import functools

import jax
import jax.numpy as jnp
from jax import lax
from jax.experimental import pallas as pl
from jax.experimental.pallas import tpu as pltpu
from jax.experimental.pallas import tpu_sc as plsc

VOCAB = 100000
DIM = 128
B = 16384
K = 20
J = K + 1
LANES = 16
NSEG = DIM // LANES

NC = 2
NS = 16
NW = NC * NS

GCH = 112


def _sc_body(cb, rpw, c_hbm, np_hbm, cen_hbm, ctx_hbm, out_hbm,
             ce0, cx0, sem0, out0, osem0,
             ce1, cx1, sem1, out1, osem1,
             cidx_all, npidx_all):
    nchunk = rpw // cb
    ng = (cb * J) // GCH
    osz = cb * J * LANES
    wid = lax.axis_index("s") * NC + lax.axis_index("c")
    pltpu.sync_copy(c_hbm.at[pl.ds(wid * rpw, rpw)], cidx_all)
    pltpu.sync_copy(np_hbm.at[pl.ds(wid * rpw * J, rpw * J)], npidx_all)

    bufs = ((ce0, cx0, sem0, out0, osem0),
            (ce1, cx1, sem1, out1, osem1))

    def issue(t, buf):
        ce_v, cx_v, sem = buf[0], buf[1], buf[2]
        pltpu.async_copy(
            cen_hbm.at[cidx_all.at[pl.ds(t * cb, cb)]], ce_v, sem
        )
        for g in range(ng):
            pltpu.async_copy(
                ctx_hbm.at[npidx_all.at[pl.ds(t * cb * J + g * GCH, GCH)]],
                cx_v.at[pl.ds(g * GCH, GCH)],
                sem,
            )

    def wait_g(t, buf):
        ce_v, cx_v, sem = buf[0], buf[1], buf[2]
        pltpu.make_async_copy(
            cen_hbm.at[cidx_all.at[pl.ds(t * cb, cb)]], ce_v, sem
        ).wait()
        pltpu.make_async_copy(
            ctx_hbm.at[npidx_all.at[pl.ds(t * cb * J, cb * J)]], cx_v, sem
        ).wait()

    def wait_store(buf):
        out_v, osem = buf[3], buf[4]
        pltpu.make_async_copy(
            out_v, out_hbm.at[pl.ds(0, osz)], osem
        ).wait()

    def compute(t, buf):
        ce_v, cx_v, out_v, osem = buf[0], buf[1], buf[3], buf[4]

        def one_row(b):
            ce = [ce_v[b, pl.ds(LANES * i, LANES)] for i in range(NSEG)]

            def ctx_row(j):
                r = b * J + j
                return [
                    cx_v[r, pl.ds(LANES * i, LANES)] for i in range(NSEG)
                ]

            cur = ctx_row(0)
            for j in range(J):
                nxt = ctx_row(j + 1) if j + 1 < J else None
                prods = [ce[i] * cur[i] for i in range(NSEG)]
                while len(prods) > 1:
                    prods = [
                        prods[q] + prods[q + 1]
                        for q in range(0, len(prods), 2)
                    ]
                out_v[pl.ds((b * J + j) * LANES, LANES)] = prods[0]
                cur = nxt

        def row_body(b, c2):
            one_row(b)
            return c2

        lax.fori_loop(0, cb, row_body, 0, unroll=False)
        base = (wid * rpw + t * cb) * J * LANES
        pltpu.async_copy(out_v, out_hbm.at[pl.ds(base, osz)], osem)

    issue(0, bufs[0])
    issue(1, bufs[1])
    wait_g(0, bufs[0])
    compute(0, bufs[0])
    issue(2, bufs[0])
    wait_g(1, bufs[1])
    compute(1, bufs[1])
    issue(3, bufs[1])

    def pair_body(g, carry):
        t0 = 2 * g
        wait_g(t0, bufs[0])
        wait_store(bufs[0])
        compute(t0, bufs[0])
        issue(lax.rem(t0 + 2, nchunk), bufs[0])
        wait_g(t0 + 1, bufs[1])
        wait_store(bufs[1])
        compute(t0 + 1, bufs[1])
        issue(lax.rem(t0 + 3, nchunk), bufs[1])
        return carry

    lax.fori_loop(1, nchunk // 2, pair_body, 0, unroll=False)
    wait_g(0, bufs[0])
    wait_g(1, bufs[1])
    wait_store(bufs[0])
    wait_store(bufs[1])


def _make_sc_dots(b_total, cb, interpret=False):
    rpw = b_total // NW
    buf = [
        pltpu.VMEM((cb, DIM), jnp.float32),
        pltpu.VMEM((cb * J, DIM), jnp.float32),
        pltpu.SemaphoreType.DMA,
        pltpu.VMEM((cb * J * LANES,), jnp.float32),
        pltpu.SemaphoreType.DMA,
    ]
    return functools.partial(
        pl.kernel,
        out_type=jax.ShapeDtypeStruct((b_total * J * LANES,), jnp.float32),
        mesh=plsc.VectorSubcoreMesh(
            core_axis_name="c", subcore_axis_name="s",
            num_cores=NC, num_subcores=NS,
        ),
        scratch_types=buf + buf + [
            pltpu.VMEM((rpw,), jnp.int32),
            pltpu.VMEM((rpw * J,), jnp.int32),
        ],
        compiler_params=pltpu.CompilerParams(needs_layout_passes=False),
        interpret=interpret,
    )(functools.partial(_sc_body, cb, rpw))


def _loss_body(bk, x_ref, o_ref):
    x = x_ref[...]
    l_ = lax.broadcasted_iota(jnp.int32, (128, 128), 0)
    c_ = lax.broadcasted_iota(jnp.int32, (128, 128), 1)
    m = (l_ // LANES == c_).astype(jnp.float32)
    s = jnp.dot(x, m, preferred_element_type=jnp.float32)
    row = lax.broadcasted_iota(jnp.int32, s.shape, 0)
    col = lax.broadcasted_iota(jnp.int32, s.shape, 1)
    j = (row * 8 + col) % J
    valid = col < 8
    sign = jnp.where(j == K, 1.0, -1.0).astype(jnp.float32)
    w = jnp.where(
        valid,
        jnp.where(j == K, 1.0 / bk, 1.0 / (bk * K)),
        0.0,
    ).astype(jnp.float32)
    t = sign * s
    ls = jnp.minimum(t, 0.0) - jnp.log1p(jnp.exp(-jnp.abs(t)))
    o_ref[0, 0] = -jnp.sum(w * ls)


def _loss_from_dots(dots, bk, interpret=False):
    x = dots.reshape(bk * J * LANES // 128, 128)
    out = pl.pallas_call(
        functools.partial(_loss_body, bk),
        out_shape=jax.ShapeDtypeStruct((1, 1), jnp.float32),
        out_specs=pl.BlockSpec(memory_space=pltpu.SMEM),
        interpret=interpret,
    )(x)
    return out[0, 0]


@jax.jit
def kernel(c, p, n, center_w, context_w):
    c = c.astype(jnp.int32)
    np_idx = jnp.concatenate(
        [n.astype(jnp.int32), p.astype(jnp.int32)[:, None]], axis=1
    ).reshape(-1)
    dots = _make_sc_dots(B, 16)(c, np_idx, center_w, context_w)
    return _loss_from_dots(dots, B)
